# baseline (speedup 1.0000x reference)
import functools

import jax
import jax.numpy as jnp
from jax.experimental import pallas as pl
from jax.experimental.pallas import tpu as pltpu


def _phase_kernel(ctx_ref, w_ref, b_ref, out_ref, m_ref, s_ref, *, nk, kt, k_total):
    p = pl.program_id(0)
    k = pl.program_id(1)

    logits = jax.lax.dot_general(
        ctx_ref[...],
        w_ref[...],
        dimension_numbers=(((1,), (0,)), ((), ())),
        preferred_element_type=jnp.float32,
    ) + b_ref[...]

    @pl.when(p == 0)
    def _pass1():
        @pl.when(k == 0)
        def _init():
            m_ref[...] = jnp.full_like(m_ref[...], -jnp.inf)
            s_ref[...] = jnp.zeros_like(s_ref[...])

        col = k * kt + jax.lax.broadcasted_iota(jnp.int32, (1, kt), 1)
        lm = jnp.where(col < k_total, logits, -jnp.inf)
        t_max = jnp.max(lm, axis=1, keepdims=True)
        m_old = m_ref[...]
        m_new = jnp.maximum(m_old, t_max)
        s_ref[...] = s_ref[...] * jnp.exp(m_old - m_new) + jnp.sum(
            jnp.exp(lm - m_new), axis=1, keepdims=True
        )
        m_ref[...] = m_new

        @pl.when(k == nk - 1)
        def _finalize():
            m_ref[...] = m_ref[...] + jnp.log(s_ref[...])

    @pl.when(p == 1)
    def _pass2():
        out_ref[...] = logits - m_ref[...]


@jax.jit
def kernel(context, W, b):
    B, D = context.shape
    K = W.shape[1]
    KT = 1024
    NK = -(-K // KT)
    b2 = b.reshape(1, K)

    return pl.pallas_call(
        functools.partial(_phase_kernel, nk=NK, kt=KT, k_total=K),
        grid=(2, NK),
        in_specs=[
            pl.BlockSpec((B, D), lambda p, k: (0, 0)),
            pl.BlockSpec((D, KT), lambda p, k: (0, k)),
            pl.BlockSpec((1, KT), lambda p, k: (0, k)),
        ],
        out_specs=pl.BlockSpec((B, KT), lambda p, k: (0, k * p)),
        out_shape=jax.ShapeDtypeStruct((B, K), jnp.float32),
        scratch_shapes=[
            pltpu.VMEM((B, 1), jnp.float32),
            pltpu.VMEM((B, 1), jnp.float32),
        ],
        compiler_params=pltpu.CompilerParams(
            dimension_semantics=("arbitrary", "arbitrary"),
        ),
    )(context, W, b2)
